# baseline (speedup 1.0000x reference)
import jax
import jax.numpy as jnp
from jax.experimental import pallas as pl


B, N, M, K = 32, 2048, 512, 8
BOTTLENECK = 1024


def _mlp_kernel(gf_ref, w1_ref, b1_ref, g1_ref, be1_ref,
                w2_ref, b2_ref, g2_ref, be2_ref,
                w3_ref, b3_ref, g3_ref, be3_ref,
                w4_ref, b4_ref, y_ref):
    def bn_relu(y, g, be):
        mean = jnp.mean(y, axis=0, keepdims=True)
        var = jnp.mean((y - mean) * (y - mean), axis=0, keepdims=True)
        return jax.nn.relu((y - mean) * jax.lax.rsqrt(var + 1e-5) * g + be)

    y = jnp.dot(gf_ref[...], w1_ref[...], preferred_element_type=jnp.float32)
    y = bn_relu(y + b1_ref[...], g1_ref[...], be1_ref[...])
    y = jnp.dot(y, w2_ref[...], preferred_element_type=jnp.float32)
    y = bn_relu(y + b2_ref[...], g2_ref[...], be2_ref[...])
    y = jnp.dot(y, w3_ref[...], preferred_element_type=jnp.float32)
    y = bn_relu(y + b3_ref[...], g3_ref[...], be3_ref[...])
    y = jnp.dot(y, w4_ref[...], preferred_element_type=jnp.float32)
    y_ref[...] = y + b4_ref[...]


def _proj_kernel(q_ref, xt_ref, x_ref, isig_ref, out_ref):
    q = q_ref[0]
    pt = xt_ref[0]
    p = x_ref[0]
    inv_sigma = isig_ref[0, 0]

    qp = jnp.dot(q, pt, preferred_element_type=jnp.float32)
    q2 = jnp.sum(q * q, axis=1, keepdims=True)
    p2 = jnp.sum(pt * pt, axis=0, keepdims=True)
    d2sel = q2 - 2.0 * qp + p2

    dmin = jnp.min(d2sel, axis=1, keepdims=True)
    t = dmin
    for _ in range(K - 1):
        t = jnp.min(jnp.where(d2sel <= t, jnp.inf, d2sel), axis=1,
                    keepdims=True)

    w = jnp.where(d2sel <= t, jnp.exp((dmin - d2sel) * inv_sigma), 0.0)

    p4 = jnp.concatenate([p, jnp.ones((N, 1), jnp.float32)], axis=1)
    res = jnp.dot(w, p4, precision=jax.lax.Precision.HIGHEST,
                  preferred_element_type=jnp.float32)
    out_ref[0] = res[:, 0:3] / res[:, 3:4]


def kernel(x, global_feat, W1, b1, g1, be1, W2, b2, g2, be2, W3, b3, g3, be3,
           W4, b4, temperature):
    f32 = jnp.float32

    y = pl.pallas_call(
        _mlp_kernel,
        out_shape=jax.ShapeDtypeStruct((B, 3 * M), f32),
    )(global_feat, W1.T, b1.reshape(1, -1), g1.reshape(1, -1),
      be1.reshape(1, -1), W2.T, b2.reshape(1, -1), g2.reshape(1, -1),
      be2.reshape(1, -1), W3.T, b3.reshape(1, -1), g3.reshape(1, -1),
      be3.reshape(1, -1), W4.T, b4.reshape(1, -1))

    generated = jnp.transpose(y.reshape(B, 3, M), (0, 2, 1))
    xt = jnp.transpose(x, (0, 2, 1))
    sigma = jnp.maximum(temperature * temperature, 0.01)
    inv_sigma = (1.0 / sigma).reshape(1, 1).astype(f32)

    proj = pl.pallas_call(
        _proj_kernel,
        grid=(B,),
        in_specs=[
            pl.BlockSpec((1, M, 3), lambda b: (b, 0, 0)),
            pl.BlockSpec((1, 3, N), lambda b: (b, 0, 0)),
            pl.BlockSpec((1, N, 3), lambda b: (b, 0, 0)),
            pl.BlockSpec((1, 1), lambda b: (0, 0)),
        ],
        out_specs=pl.BlockSpec((1, M, 3), lambda b: (b, 0, 0)),
        out_shape=jax.ShapeDtypeStruct((B, M, 3), f32),
    )(generated, xt, x, inv_sigma)

    return generated, proj


# speedup vs baseline: 1.5300x; 1.5300x over previous
import jax
import jax.numpy as jnp
from jax.experimental import pallas as pl


B, N, M, K = 32, 2048, 512, 8
BOTTLENECK = 1024


def _mlp_kernel(gf_ref, w1_ref, b1_ref, g1_ref, be1_ref,
                w2_ref, b2_ref, g2_ref, be2_ref,
                w3_ref, b3_ref, g3_ref, be3_ref,
                w4_ref, b4_ref, y_ref):
    def bn_relu(y, g, be):
        mean = jnp.mean(y, axis=0, keepdims=True)
        var = jnp.mean((y - mean) * (y - mean), axis=0, keepdims=True)
        return jax.nn.relu((y - mean) * jax.lax.rsqrt(var + 1e-5) * g + be)

    y = jnp.dot(gf_ref[...], w1_ref[...], preferred_element_type=jnp.float32)
    y = bn_relu(y + b1_ref[...], g1_ref[...], be1_ref[...])
    y = jnp.dot(y, w2_ref[...], preferred_element_type=jnp.float32)
    y = bn_relu(y + b2_ref[...], g2_ref[...], be2_ref[...])
    y = jnp.dot(y, w3_ref[...], preferred_element_type=jnp.float32)
    y = bn_relu(y + b3_ref[...], g3_ref[...], be3_ref[...])
    y = jnp.dot(y, w4_ref[...], preferred_element_type=jnp.float32)
    y_ref[...] = y + b4_ref[...]


def _proj_kernel(q_ref, xt_ref, isig_ref, out_ref):
    q = q_ref[0]
    pt = xt_ref[0]
    inv_sigma = isig_ref[0, 0]

    qp = jnp.dot(q, pt, preferred_element_type=jnp.float32)
    q2 = jnp.sum(q * q, axis=1, keepdims=True)
    p2 = jnp.sum(pt * pt, axis=0, keepdims=True)
    d2sel = q2 - 2.0 * qp + p2

    dmin = jnp.min(d2sel, axis=1, keepdims=True)
    t = dmin
    for _ in range(K - 1):
        t = jnp.min(jnp.where(d2sel <= t, jnp.inf, d2sel), axis=1,
                    keepdims=True)

    w = jnp.where(d2sel <= t, jnp.exp((dmin - d2sel) * inv_sigma), 0.0)

    px = pt[0:1, :]
    py = pt[1:2, :]
    pz = pt[2:3, :]
    wsum = jnp.sum(w, axis=1, keepdims=True)
    ox = jnp.sum(w * px, axis=1, keepdims=True)
    oy = jnp.sum(w * py, axis=1, keepdims=True)
    oz = jnp.sum(w * pz, axis=1, keepdims=True)
    out_ref[0] = jnp.concatenate([ox, oy, oz], axis=1) / wsum


def kernel(x, global_feat, W1, b1, g1, be1, W2, b2, g2, be2, W3, b3, g3, be3,
           W4, b4, temperature):
    f32 = jnp.float32

    y = pl.pallas_call(
        _mlp_kernel,
        out_shape=jax.ShapeDtypeStruct((B, 3 * M), f32),
    )(global_feat, W1.T, b1.reshape(1, -1), g1.reshape(1, -1),
      be1.reshape(1, -1), W2.T, b2.reshape(1, -1), g2.reshape(1, -1),
      be2.reshape(1, -1), W3.T, b3.reshape(1, -1), g3.reshape(1, -1),
      be3.reshape(1, -1), W4.T, b4.reshape(1, -1))

    generated = jnp.transpose(y.reshape(B, 3, M), (0, 2, 1))
    xt = jnp.transpose(x, (0, 2, 1))
    sigma = jnp.maximum(temperature * temperature, 0.01)
    inv_sigma = (1.0 / sigma).reshape(1, 1).astype(f32)

    proj = pl.pallas_call(
        _proj_kernel,
        grid=(B,),
        in_specs=[
            pl.BlockSpec((1, M, 3), lambda b: (b, 0, 0)),
            pl.BlockSpec((1, 3, N), lambda b: (b, 0, 0)),
            pl.BlockSpec((1, 1), lambda b: (0, 0)),
        ],
        out_specs=pl.BlockSpec((1, M, 3), lambda b: (b, 0, 0)),
        out_shape=jax.ShapeDtypeStruct((B, M, 3), f32),
    )(generated, xt, inv_sigma)

    return generated, proj


# speedup vs baseline: 1.5481x; 1.0118x over previous
import jax
import jax.numpy as jnp
from jax.experimental import pallas as pl


B, N, M, K = 32, 2048, 512, 8
BOTTLENECK = 1024


def _mlp_kernel(gf_ref, w1_ref, b1_ref, g1_ref, be1_ref,
                w2_ref, b2_ref, g2_ref, be2_ref,
                w3_ref, b3_ref, g3_ref, be3_ref,
                w4_ref, b4_ref, y_ref):
    def bn_relu(y, g, be):
        mean = jnp.mean(y, axis=0, keepdims=True)
        var = jnp.mean((y - mean) * (y - mean), axis=0, keepdims=True)
        return jax.nn.relu((y - mean) * jax.lax.rsqrt(var + 1e-5) * g + be)

    y = jnp.dot(gf_ref[...], w1_ref[...], preferred_element_type=jnp.float32)
    y = bn_relu(y + b1_ref[...], g1_ref[...], be1_ref[...])
    y = jnp.dot(y, w2_ref[...], preferred_element_type=jnp.float32)
    y = bn_relu(y + b2_ref[...], g2_ref[...], be2_ref[...])
    y = jnp.dot(y, w3_ref[...], preferred_element_type=jnp.float32)
    y = bn_relu(y + b3_ref[...], g3_ref[...], be3_ref[...])
    y = jnp.dot(y, w4_ref[...], preferred_element_type=jnp.float32)
    y_ref[...] = y + b4_ref[...]


def _proj_kernel(q_ref, xt_ref, isig_ref, out_ref):
    q = q_ref[0]
    pt = xt_ref[0]
    inv_sigma = isig_ref[0, 0]

    qp2 = jnp.dot(-2.0 * q, pt, preferred_element_type=jnp.float32)
    q2 = jnp.sum(q * q, axis=1, keepdims=True)
    p2 = jnp.sum(pt * pt, axis=0, keepdims=True)
    d2sel = (q2 + qp2) + p2

    dmin = jnp.min(d2sel, axis=1, keepdims=True)
    t = dmin
    for _ in range(K - 1):
        t = jnp.min(jnp.where(d2sel <= t, jnp.inf, d2sel), axis=1,
                    keepdims=True)

    c2 = inv_sigma * 1.4426950408889634
    w = jnp.where(d2sel <= t, jnp.exp2((dmin - d2sel) * c2), 0.0)

    px = pt[0:1, :]
    py = pt[1:2, :]
    pz = pt[2:3, :]
    wsum = jnp.sum(w, axis=1, keepdims=True)
    ox = jnp.sum(w * px, axis=1, keepdims=True)
    oy = jnp.sum(w * py, axis=1, keepdims=True)
    oz = jnp.sum(w * pz, axis=1, keepdims=True)
    out_ref[0] = jnp.concatenate([ox, oy, oz], axis=1) / wsum


def kernel(x, global_feat, W1, b1, g1, be1, W2, b2, g2, be2, W3, b3, g3, be3,
           W4, b4, temperature):
    f32 = jnp.float32

    y = pl.pallas_call(
        _mlp_kernel,
        out_shape=jax.ShapeDtypeStruct((B, 3 * M), f32),
    )(global_feat, W1.T, b1.reshape(1, -1), g1.reshape(1, -1),
      be1.reshape(1, -1), W2.T, b2.reshape(1, -1), g2.reshape(1, -1),
      be2.reshape(1, -1), W3.T, b3.reshape(1, -1), g3.reshape(1, -1),
      be3.reshape(1, -1), W4.T, b4.reshape(1, -1))

    generated = jnp.transpose(y.reshape(B, 3, M), (0, 2, 1))
    xt = jnp.transpose(x, (0, 2, 1))
    sigma = jnp.maximum(temperature * temperature, 0.01)
    inv_sigma = (1.0 / sigma).reshape(1, 1).astype(f32)

    proj = pl.pallas_call(
        _proj_kernel,
        grid=(B,),
        in_specs=[
            pl.BlockSpec((1, M, 3), lambda b: (b, 0, 0)),
            pl.BlockSpec((1, 3, N), lambda b: (b, 0, 0)),
            pl.BlockSpec((1, 1), lambda b: (0, 0)),
        ],
        out_specs=pl.BlockSpec((1, M, 3), lambda b: (b, 0, 0)),
        out_shape=jax.ShapeDtypeStruct((B, M, 3), f32),
    )(generated, xt, inv_sigma)

    return generated, proj
